# baseline (speedup 1.0000x reference)
import functools

import jax
import jax.numpy as jnp
import numpy as np
from jax import lax
from jax.experimental import pallas as pl
from jax.experimental.pallas import tpu as pltpu
from jax.experimental.pallas import tpu_sc as plsc

B = 16384
L = 200
V = 10000
D = 100
DPB = 128
RW = 64
N_OUT = 256

NC, NS = 2, 16
NW = NC * NS
ROWS_PER_W = B // NW
CB = 4
IDX_ROWS = 2 * CB
CHUNKS = ROWS_PER_W // CB
LANES = 16
I32_CH = RW // LANES
GRP = 10
NGRP = L // GRP

GROUP = 16
OUTER = CHUNKS // 2

_HI_MASK = np.int32(-65536)


def _widen_lo(v_i32):
    return plsc.bitcast(lax.shift_left(v_i32, 16), jnp.float32)


def _widen_hi(v_i32):
    return plsc.bitcast(lax.bitwise_and(v_i32, _HI_MASK), jnp.float32)


def _sc_bag(x_hbm, table_hbm, out_hbm, idx0, idx1, rows0, rows1, stage,
            gsem0, gsem1, isem):
    wid = lax.axis_index("s") * NC + lax.axis_index("c")
    ibase0 = wid * (ROWS_PER_W * 2)
    obase0 = wid * ROWS_PER_W
    idx_b = (idx0, idx1)
    rows_b = (rows0, rows1)
    gsem_b = (gsem0, gsem1)

    def fire_gathers(p):
        for j in range(IDX_ROWS):
            pltpu.async_copy(
                table_hbm.at[idx_b[p].at[j]],
                rows_b[p].at[pl.ds(j * 100, 100)],
                gsem_b[p],
            )

    def wait_gathers(p):
        for j in range(IDX_ROWS):
            pltpu.make_async_copy(
                table_hbm.at[idx_b[p].at[j]],
                rows_b[p].at[pl.ds(j * 100, 100)],
                gsem_b[p],
            ).wait()

    def fire_idx(p, i):
        pltpu.async_copy(
            x_hbm.at[pl.ds(ibase0 + i * IDX_ROWS, IDX_ROWS)], idx_b[p], isem)

    def wait_idx(p, i):
        pltpu.make_async_copy(
            x_hbm.at[pl.ds(ibase0 + i * IDX_ROWS, IDX_ROWS)], idx_b[p], isem,
        ).wait()

    pltpu.sync_copy(x_hbm.at[pl.ds(ibase0, IDX_ROWS)], idx0)
    fire_gathers(0)
    fire_idx(1, 1)

    def outer_body(go, _):
        for sub in range(2):
            i = 2 * go + sub
            p = sub
            q = 1 - sub
            wait_gathers(p)
            @pl.when(i < CHUNKS - 1)
            def _():
                wait_idx(q, i + 1)
                fire_gathers(q)

            @pl.when(i < CHUNKS - 2)
            def _():
                fire_idx(p, i + 2)

            srow = (i % GROUP) * CB
            for rb in range(CB):
                def grp_body(g, facc):
                    bacc = [jnp.zeros((2 * LANES,), jnp.bfloat16)
                            for _ in range(I32_CH)]
                    base = rb * L + g * GRP
                    for r in range(GRP):
                        for c in range(I32_CH):
                            v = rows_b[p][base + r, pl.ds(c * LANES, LANES)]
                            bacc[c] = bacc[c] + plsc.bitcast(v, jnp.bfloat16)
                    out = []
                    for c in range(I32_CH):
                        pv = plsc.bitcast(bacc[c], jnp.int32)
                        out.append(facc[2 * c] + _widen_lo(pv))
                        out.append(facc[2 * c + 1] + _widen_hi(pv))
                    return tuple(out)

                facc = lax.fori_loop(
                    0, NGRP, grp_body,
                    tuple(jnp.zeros((LANES,), jnp.float32)
                          for _ in range(2 * I32_CH)),
                )
                for c in range(2 * I32_CH):
                    stage[srow + rb, pl.ds(c * LANES, LANES)] = facc[c]
        @pl.when(go % (GROUP // 2) == (GROUP // 2) - 1)
        def _():
            grp = go // (GROUP // 2)
            pltpu.sync_copy(
                stage, out_hbm.at[pl.ds(obase0 + grp * (GROUP * CB),
                                        GROUP * CB)])
        return 0

    lax.fori_loop(0, OUTER, outer_body, 0)


def _pack_body(t_ref, o_ref):
    u = jax.lax.bitcast_convert_type(t_ref[...], jnp.uint32)
    lo = u[:, :64]
    hi = jnp.pad(u[:, 64:100], ((0, 0), (0, 28)))

    def _rne(v):
        return (v + jnp.uint32(0x7FFF) + ((v >> 16) & jnp.uint32(1))) >> 16

    w = _rne(lo) | (_rne(hi) << 16)
    o_ref[...] = jax.lax.bitcast_convert_type(w, jnp.int32)


def _mm_body(p_ref, w_ref, b_ref, o_ref):
    o_ref[...] = jnp.maximum(
        jnp.dot(p_ref[...], w_ref[...], preferred_element_type=jnp.float32)
        + b_ref[...],
        0.0,
    )


def _mk_perm():
    perm = np.zeros(DPB, np.int64)
    for c in range(DPB // 32):
        for k in range(16):
            j = 16 * c + k
            perm[32 * c + k] = j
            perm[32 * c + 16 + k] = j + 64 if j + 64 < D else 0
    return perm


_PERM = _mk_perm()


def kernel(x, table, W, b):
    xr = x.reshape(B * 2, 100).astype(jnp.int32)
    PBM = 1000
    tpk = pl.pallas_call(
        _pack_body,
        grid=(V // PBM,),
        in_specs=[pl.BlockSpec((PBM, D), lambda i: (i, 0))],
        out_specs=pl.BlockSpec((PBM, RW), lambda i: (i, 0)),
        out_shape=jax.ShapeDtypeStruct((V, RW), jnp.int32),
    )(table)
    Wp = jnp.pad(W * (1.0 / L), ((0, DPB - D), (0, 0)))[_PERM, :]
    b2 = b.reshape(1, N_OUT)

    mesh = plsc.VectorSubcoreMesh(core_axis_name="c", subcore_axis_name="s")
    sc_fn = functools.partial(
        pl.kernel,
        mesh=mesh,
        compiler_params=pltpu.CompilerParams(use_tc_tiling_on_sc=False,
                                             needs_layout_passes=False),
        out_type=jax.ShapeDtypeStruct((B, DPB), jnp.float32),
        scratch_types=[
            pltpu.VMEM((IDX_ROWS, 100), jnp.int32),
            pltpu.VMEM((IDX_ROWS, 100), jnp.int32),
            pltpu.VMEM((CB * L, RW), jnp.int32),
            pltpu.VMEM((CB * L, RW), jnp.int32),
            pltpu.VMEM((GROUP * CB, DPB), jnp.float32),
            pltpu.SemaphoreType.DMA,
            pltpu.SemaphoreType.DMA,
            pltpu.SemaphoreType.DMA,
        ],
    )(_sc_bag)
    pooled = sc_fn(xr, tpk)

    BM = 1024
    out = pl.pallas_call(
        _mm_body,
        grid=(B // BM,),
        in_specs=[
            pl.BlockSpec((BM, DPB), lambda i: (i, 0)),
            pl.BlockSpec((DPB, N_OUT), lambda i: (0, 0)),
            pl.BlockSpec((1, N_OUT), lambda i: (0, 0)),
        ],
        out_specs=pl.BlockSpec((BM, N_OUT), lambda i: (i, 0)),
        out_shape=jax.ShapeDtypeStruct((B, N_OUT), jnp.float32),
    )(pooled, Wp, b2)
    return out


# speedup vs baseline: 1.0591x; 1.0591x over previous
import functools

import jax
import jax.numpy as jnp
import numpy as np
from jax import lax
from jax.experimental import pallas as pl
from jax.experimental.pallas import tpu as pltpu
from jax.experimental.pallas import tpu_sc as plsc

B = 16384
L = 200
V = 10000
D = 100
DPB = 128
RW = 64
N_OUT = 256

NC, NS = 2, 16
NW = NC * NS
ROWS_PER_W = B // NW
CB = 4
IDX_ROWS = 2 * CB
CHUNKS = ROWS_PER_W // CB
LANES = 16
I32_CH = RW // LANES
GRP = 10
NGRP = L // GRP

GROUP = 16
OUTER = CHUNKS // 2

_HI_MASK = np.int32(-65536)


def _widen_lo(v_i32):
    return plsc.bitcast(lax.shift_left(v_i32, 16), jnp.float32)


def _widen_hi(v_i32):
    return plsc.bitcast(lax.bitwise_and(v_i32, _HI_MASK), jnp.float32)


def _sc_bag(x_hbm, table_hbm, out_hbm, idx0a, idx0b, idx1a, idx1b,
            rows0, rows1, stage, gsem0, gsem1, isem):
    wid = lax.axis_index("s") * NC + lax.axis_index("c")
    obase0 = wid * ROWS_PER_W
    idx_b = ((idx0a, idx0b), (idx1a, idx1b))
    rows_b = (rows0, rows1)
    gsem_b = (gsem0, gsem1)
    SPLITS = ((0, 104), (104, 96))

    def fire_gathers(p):
        for rb in range(CB):
            for h in range(2):
                off, n = SPLITS[h]
                pltpu.async_copy(
                    table_hbm.at[idx_b[p][h].at[rb]],
                    rows_b[p].at[pl.ds(rb * L + off, n)],
                    gsem_b[p],
                )

    def wait_gathers(p):
        for rb in range(CB):
            for h in range(2):
                off, n = SPLITS[h]
                pltpu.make_async_copy(
                    table_hbm.at[idx_b[p][h].at[rb]],
                    rows_b[p].at[pl.ds(rb * L + off, n)],
                    gsem_b[p],
                ).wait()

    def _idx_copies(p, i):
        rowbase = obase0 + i * CB
        return [
            pltpu.make_async_copy(
                x_hbm.at[pl.ds(rowbase, CB), pl.ds(SPLITS[h][0], SPLITS[h][1])],
                idx_b[p][h], isem)
            for h in range(2)
        ]

    def fire_idx(p, i):
        for cp in _idx_copies(p, i):
            cp.start()

    def wait_idx(p, i):
        for cp in _idx_copies(p, i):
            cp.wait()

    fire_idx(0, 0)
    wait_idx(0, 0)
    fire_gathers(0)
    fire_idx(1, 1)

    def outer_body(go, _):
        for sub in range(2):
            i = 2 * go + sub
            p = sub
            q = 1 - sub
            wait_gathers(p)
            @pl.when(i < CHUNKS - 1)
            def _():
                wait_idx(q, i + 1)
                fire_gathers(q)

            @pl.when(i < CHUNKS - 2)
            def _():
                fire_idx(p, i + 2)

            srow = (i % GROUP) * CB
            for rb in range(CB):
                def grp_body(g, facc):
                    bacc = [jnp.zeros((2 * LANES,), jnp.bfloat16)
                            for _ in range(I32_CH)]
                    base = rb * L + g * GRP
                    for r in range(GRP):
                        for c in range(I32_CH):
                            v = rows_b[p][base + r, pl.ds(c * LANES, LANES)]
                            bacc[c] = bacc[c] + plsc.bitcast(v, jnp.bfloat16)
                    out = []
                    for c in range(I32_CH):
                        pv = plsc.bitcast(bacc[c], jnp.int32)
                        out.append(facc[2 * c] + _widen_lo(pv))
                        out.append(facc[2 * c + 1] + _widen_hi(pv))
                    return tuple(out)

                facc = lax.fori_loop(
                    0, NGRP, grp_body,
                    tuple(jnp.zeros((LANES,), jnp.float32)
                          for _ in range(2 * I32_CH)),
                )
                for c in range(2 * I32_CH):
                    stage[srow + rb, pl.ds(c * LANES, LANES)] = facc[c]
        @pl.when(go % (GROUP // 2) == (GROUP // 2) - 1)
        def _():
            grp = go // (GROUP // 2)
            pltpu.sync_copy(
                stage, out_hbm.at[pl.ds(obase0 + grp * (GROUP * CB),
                                        GROUP * CB)])
        return 0

    lax.fori_loop(0, OUTER, outer_body, 0)


def _pack_body(t_ref, o_ref):
    u = jax.lax.bitcast_convert_type(t_ref[...], jnp.uint32)
    lo = u[:, :64]
    hi = jnp.pad(u[:, 64:100], ((0, 0), (0, 28)))

    def _rne(v):
        return (v + jnp.uint32(0x7FFF) + ((v >> 16) & jnp.uint32(1))) >> 16

    w = _rne(lo) | (_rne(hi) << 16)
    o_ref[...] = jax.lax.bitcast_convert_type(w, jnp.int32)


def _mm_body(p_ref, w_ref, b_ref, o_ref):
    o_ref[...] = jnp.maximum(
        jnp.dot(p_ref[...], w_ref[...], preferred_element_type=jnp.float32)
        + b_ref[...],
        0.0,
    )


def _mk_perm():
    perm = np.zeros(DPB, np.int64)
    for c in range(DPB // 32):
        for k in range(16):
            j = 16 * c + k
            perm[32 * c + k] = j
            perm[32 * c + 16 + k] = j + 64 if j + 64 < D else 0
    return perm


_PERM = _mk_perm()


def kernel(x, table, W, b):
    PBM = 1000
    tpk = pl.pallas_call(
        _pack_body,
        grid=(V // PBM,),
        in_specs=[pl.BlockSpec((PBM, D), lambda i: (i, 0))],
        out_specs=pl.BlockSpec((PBM, RW), lambda i: (i, 0)),
        out_shape=jax.ShapeDtypeStruct((V, RW), jnp.int32),
    )(table)
    Wp = jnp.pad(W * (1.0 / L), ((0, DPB - D), (0, 0)))[_PERM, :]
    b2 = b.reshape(1, N_OUT)

    mesh = plsc.VectorSubcoreMesh(core_axis_name="c", subcore_axis_name="s")
    sc_fn = functools.partial(
        pl.kernel,
        mesh=mesh,
        compiler_params=pltpu.CompilerParams(use_tc_tiling_on_sc=False,
                                             needs_layout_passes=False),
        out_type=jax.ShapeDtypeStruct((B, DPB), jnp.float32),
        scratch_types=[
            pltpu.VMEM((CB, 104), jnp.int32),
            pltpu.VMEM((CB, 96), jnp.int32),
            pltpu.VMEM((CB, 104), jnp.int32),
            pltpu.VMEM((CB, 96), jnp.int32),
            pltpu.VMEM((CB * L, RW), jnp.int32),
            pltpu.VMEM((CB * L, RW), jnp.int32),
            pltpu.VMEM((GROUP * CB, DPB), jnp.float32),
            pltpu.SemaphoreType.DMA,
            pltpu.SemaphoreType.DMA,
            pltpu.SemaphoreType.DMA,
        ],
    )(_sc_bag)
    pooled = sc_fn(x, tpk)

    BM = 1024
    out = pl.pallas_call(
        _mm_body,
        grid=(B // BM,),
        in_specs=[
            pl.BlockSpec((BM, DPB), lambda i: (i, 0)),
            pl.BlockSpec((DPB, N_OUT), lambda i: (0, 0)),
            pl.BlockSpec((1, N_OUT), lambda i: (0, 0)),
        ],
        out_specs=pl.BlockSpec((BM, N_OUT), lambda i: (i, 0)),
        out_shape=jax.ShapeDtypeStruct((B, N_OUT), jnp.float32),
    )(pooled, Wp, b2)
    return out
